# Initial kernel scaffold; baseline (speedup 1.0000x reference)
import jax
import jax.numpy as jnp
from jax.experimental import pallas as pl


def kernel(x, edge_index, W1, b1, W2, b2):
    raise NotImplementedError("write your pallas kernel here")


# speedup vs baseline: 9.9057x; 9.9057x over previous
import functools

import jax
import jax.numpy as jnp
from jax import lax
from jax.experimental import pallas as pl
from jax.experimental.pallas import tpu as pltpu
from jax.experimental.pallas import tpu_sc as plsc

N_NODES = 10000
N_EDGES = 320000
D_IN = 128
D_HID = 64

NC = 2
NS = 16
NW = NC * NS
EPW = N_EDGES // NW
CHUNK = 125
NCHUNK = EPW // CHUNK
STRIPE = 1000
NSTRIPE_TILES = N_NODES // STRIPE


def _mm1_body(x_ref, w_ref, o_ref):
    o_ref[...] = jnp.dot(x_ref[...], w_ref[...],
                         preferred_element_type=jnp.float32)


def _epilogue_body(y_ref, p0_ref, p1_ref, b1_ref, w2_ref, b2_ref, o_ref):
    h = y_ref[...] + p0_ref[...] + p1_ref[...] + b1_ref[...]
    h = jnp.maximum(h, 0.0)
    o_ref[...] = jnp.dot(h, w2_ref[...],
                         preferred_element_type=jnp.float32) + b2_ref[...]


def _sc_segsum_body(y_hbm, src_hbm, dst_hbm, zeros_hbm, out_hbm,
                    si_v, di_v, rows_v, agg_sh, gsem):
    c = lax.axis_index("c")
    s = lax.axis_index("s")
    w = c * NS + s
    row_base = w * NCHUNK

    pltpu.sync_copy(src_hbm.at[pl.ds(row_base, NCHUNK)], si_v)
    pltpu.sync_copy(dst_hbm.at[pl.ds(row_base, NCHUNK)], di_v)

    zbase = s * STRIPE
    @pl.when(s < NSTRIPE_TILES)
    def _():
        pltpu.sync_copy(zeros_hbm.at[pl.ds(zbase, STRIPE)],
                        agg_sh.at[pl.ds(zbase, STRIPE)])
    plsc.subcore_barrier()

    def body(i, _):
        pltpu.async_copy(y_hbm.at[si_v.at[i]], rows_v, gsem).wait()
        pltpu.sync_copy(rows_v, agg_sh.at[di_v.at[i]], add=True)
        return _

    lax.fori_loop(0, NCHUNK, body, None)

    plsc.subcore_barrier()
    obase = c * N_NODES + s * STRIPE
    @pl.when(s < NSTRIPE_TILES)
    def _():
        pltpu.sync_copy(agg_sh.at[pl.ds(zbase, STRIPE)],
                        out_hbm.at[pl.ds(obase, STRIPE)])


@jax.jit
def kernel(x, edge_index, W1, b1, W2, b2):
    ei = edge_index.astype(jnp.int32)
    src2d = ei[0].reshape(NW * NCHUNK, CHUNK)
    dst2d = ei[1].reshape(NW * NCHUNK, CHUNK)
    zeros = jnp.zeros((N_NODES, D_HID), jnp.float32)

    y = pl.pallas_call(
        _mm1_body,
        out_shape=jax.ShapeDtypeStruct((N_NODES, D_HID), jnp.float32),
    )(x, W1)

    sc_segsum = pl.kernel(
        _sc_segsum_body,
        out_type=jax.ShapeDtypeStruct((NC * N_NODES, D_HID), jnp.float32),
        mesh=plsc.VectorSubcoreMesh(core_axis_name="c", subcore_axis_name="s"),
        compiler_params=pltpu.CompilerParams(use_tc_tiling_on_sc=False),
        scratch_types=[
            pltpu.VMEM((NCHUNK, CHUNK), jnp.int32),
            pltpu.VMEM((NCHUNK, CHUNK), jnp.int32),
            pltpu.VMEM((CHUNK, D_HID), jnp.float32),
            pltpu.VMEM_SHARED((N_NODES, D_HID), jnp.float32),
            pltpu.SemaphoreType.DMA,
        ],
    )
    partials = sc_segsum(y, src2d, dst2d, zeros)
    p0 = partials[:N_NODES]
    p1 = partials[N_NODES:]

    out = pl.pallas_call(
        _epilogue_body,
        out_shape=jax.ShapeDtypeStruct((N_NODES, D_HID), jnp.float32),
    )(y, p0, p1, b1.reshape(1, D_HID), W2, b2.reshape(1, D_HID))
    return out


# speedup vs baseline: 13.1940x; 1.3320x over previous
import functools

import jax
import jax.numpy as jnp
from jax import lax
from jax.experimental import pallas as pl
from jax.experimental.pallas import tpu as pltpu
from jax.experimental.pallas import tpu_sc as plsc

N_NODES = 10000
N_EDGES = 320000
D_IN = 128
D_HID = 64

NC = 2
NS = 16
NW = NC * NS
EPW = N_EDGES // NW
CHUNK = 125
NCHUNK = EPW // CHUNK
STRIPE = 1000
NSTRIPE_TILES = N_NODES // STRIPE


def _mm1_body(x_ref, w_ref, o_ref):
    o_ref[...] = jnp.dot(x_ref[...], w_ref[...],
                         preferred_element_type=jnp.float32)


def _epilogue_body(y_ref, p0_ref, p1_ref, b1_ref, w2_ref, b2_ref, o_ref):
    h = y_ref[...] + p0_ref[...] + p1_ref[...] + b1_ref[...]
    h = jnp.maximum(h, 0.0)
    o_ref[...] = jnp.dot(h, w2_ref[...],
                         preferred_element_type=jnp.float32) + b2_ref[...]


def _sc_segsum_body(y_hbm, src_hbm, dst_hbm, zeros_hbm, out_hbm,
                    si_v, di_v, rows_a, rows_b, agg_sh, sem_a, sem_b):
    c = lax.axis_index("c")
    s = lax.axis_index("s")
    w = c * NS + s
    row_base = w * NCHUNK

    pltpu.sync_copy(src_hbm.at[pl.ds(row_base, NCHUNK)], si_v)
    pltpu.sync_copy(dst_hbm.at[pl.ds(row_base, NCHUNK)], di_v)

    zbase = s * STRIPE
    @pl.when(s < NSTRIPE_TILES)
    def _():
        pltpu.sync_copy(zeros_hbm.at[pl.ds(zbase, STRIPE)],
                        agg_sh.at[pl.ds(zbase, STRIPE)])
    plsc.subcore_barrier()

    pltpu.async_copy(y_hbm.at[si_v.at[0]], rows_a, sem_a)
    pltpu.async_copy(y_hbm.at[si_v.at[1]], rows_b, sem_b)

    def drain_scatter(rows_v, sem, i):
        pltpu.make_async_copy(y_hbm.at[pl.ds(0, CHUNK)], rows_v, sem).wait()
        pltpu.sync_copy(rows_v, agg_sh.at[di_v.at[i]], add=True)

    def body(j, _):
        i = 2 * j
        drain_scatter(rows_a, sem_a, i)
        pltpu.async_copy(y_hbm.at[si_v.at[i + 2]], rows_a, sem_a)
        drain_scatter(rows_b, sem_b, i + 1)
        pltpu.async_copy(y_hbm.at[si_v.at[i + 3]], rows_b, sem_b)
        return _

    lax.fori_loop(0, NCHUNK // 2 - 1, body, None)
    drain_scatter(rows_a, sem_a, NCHUNK - 2)
    drain_scatter(rows_b, sem_b, NCHUNK - 1)

    plsc.subcore_barrier()
    obase = c * N_NODES + s * STRIPE
    @pl.when(s < NSTRIPE_TILES)
    def _():
        pltpu.sync_copy(agg_sh.at[pl.ds(zbase, STRIPE)],
                        out_hbm.at[pl.ds(obase, STRIPE)])


@jax.jit
def kernel(x, edge_index, W1, b1, W2, b2):
    ei = edge_index.astype(jnp.int32)
    src2d = ei[0].reshape(NW * NCHUNK, CHUNK)
    dst2d = ei[1].reshape(NW * NCHUNK, CHUNK)
    zeros = jnp.zeros((N_NODES, D_HID), jnp.float32)

    y = pl.pallas_call(
        _mm1_body,
        out_shape=jax.ShapeDtypeStruct((N_NODES, D_HID), jnp.float32),
    )(x, W1)

    sc_segsum = pl.kernel(
        _sc_segsum_body,
        out_type=jax.ShapeDtypeStruct((NC * N_NODES, D_HID), jnp.float32),
        mesh=plsc.VectorSubcoreMesh(core_axis_name="c", subcore_axis_name="s"),
        compiler_params=pltpu.CompilerParams(use_tc_tiling_on_sc=False),
        scratch_types=[
            pltpu.VMEM((NCHUNK, CHUNK), jnp.int32),
            pltpu.VMEM((NCHUNK, CHUNK), jnp.int32),
            pltpu.VMEM((CHUNK, D_HID), jnp.float32),
            pltpu.VMEM((CHUNK, D_HID), jnp.float32),
            pltpu.VMEM_SHARED((N_NODES, D_HID), jnp.float32),
            pltpu.SemaphoreType.DMA,
            pltpu.SemaphoreType.DMA,
        ],
    )
    partials = sc_segsum(y, src2d, dst2d, zeros)
    p0 = partials[:N_NODES]
    p1 = partials[N_NODES:]

    out = pl.pallas_call(
        _epilogue_body,
        out_shape=jax.ShapeDtypeStruct((N_NODES, D_HID), jnp.float32),
    )(y, p0, p1, b1.reshape(1, D_HID), W2, b2.reshape(1, D_HID))
    return out


# speedup vs baseline: 16.5021x; 1.2507x over previous
import functools

import jax
import jax.numpy as jnp
from jax import lax
from jax.experimental import pallas as pl
from jax.experimental.pallas import tpu as pltpu
from jax.experimental.pallas import tpu_sc as plsc

N_NODES = 10000
N_EDGES = 320000
D_IN = 128
D_HID = 64

NC = 2
NS = 16
NW = NC * NS
EPW = N_EDGES // NW
CHUNK = 80
NCHUNK = EPW // CHUNK
STRIPE = 1000
NSTRIPE_TILES = N_NODES // STRIPE


def _mm1_body(x_ref, w_ref, o_ref):
    o_ref[...] = jnp.dot(x_ref[...], w_ref[...],
                         preferred_element_type=jnp.float32)


def _epilogue_body(y_ref, p0_ref, p1_ref, b1_ref, w2_ref, b2_ref, o_ref):
    h = y_ref[...] + p0_ref[...] + p1_ref[...] + b1_ref[...]
    h = jnp.maximum(h, 0.0)
    o_ref[...] = jnp.dot(h, w2_ref[...],
                         preferred_element_type=jnp.float32) + b2_ref[...]


NBUF = 4
NPIPE = (NCHUNK - 1) // NBUF - 1


def _sc_segsum_body(ei_hbm, y_hbm, zeros_hbm, out_hbm,
                    si_v, di_v, rows, agg_sh, sems):
    c = lax.axis_index("c")
    s = lax.axis_index("s")
    w = c * NS + s
    ebase = w * EPW

    pltpu.sync_copy(ei_hbm.at[0, pl.ds(ebase, EPW)], si_v)
    pltpu.sync_copy(ei_hbm.at[1, pl.ds(ebase, EPW)], di_v)

    zbase = s * STRIPE
    @pl.when(s < NSTRIPE_TILES)
    def _():
        pltpu.sync_copy(zeros_hbm.at[pl.ds(zbase, STRIPE)],
                        agg_sh.at[pl.ds(zbase, STRIPE)])
    plsc.subcore_barrier()

    def fire(b, i):
        off = pl.multiple_of(i * CHUNK, CHUNK)
        pltpu.async_copy(y_hbm.at[si_v.at[pl.ds(off, CHUNK)]], rows[b], sems[b])

    def drain_scatter(b, i):
        off = pl.multiple_of(i * CHUNK, CHUNK)
        pltpu.make_async_copy(y_hbm.at[pl.ds(0, CHUNK)], rows[b], sems[b]).wait()
        pltpu.sync_copy(rows[b], agg_sh.at[di_v.at[pl.ds(off, CHUNK)]], add=True)

    for b in range(NBUF):
        fire(b, b)

    def body(j, _):
        i = NBUF * j
        for b in range(NBUF):
            drain_scatter(b, i + b)
            fire(b, i + b + NBUF)
        return _

    lax.fori_loop(0, NPIPE, body, None)
    base = NBUF * NPIPE
    for b in range(NBUF):
        drain_scatter(b, base + b)
    for i in range(NBUF * (NPIPE + 1), NCHUNK):
        fire(0, i)
        drain_scatter(0, i)

    plsc.subcore_barrier()
    obase = c * N_NODES + s * STRIPE
    @pl.when(s < NSTRIPE_TILES)
    def _():
        pltpu.sync_copy(agg_sh.at[pl.ds(zbase, STRIPE)],
                        out_hbm.at[pl.ds(obase, STRIPE)])


@jax.jit
def kernel(x, edge_index, W1, b1, W2, b2):
    ei = edge_index.astype(jnp.int32)
    zeros = jnp.zeros((N_NODES, D_HID), jnp.float32)

    y = pl.pallas_call(
        _mm1_body,
        out_shape=jax.ShapeDtypeStruct((N_NODES, D_HID), jnp.float32),
    )(x, W1)

    sc_segsum = pl.kernel(
        _sc_segsum_body,
        out_type=jax.ShapeDtypeStruct((NC * N_NODES, D_HID), jnp.float32),
        mesh=plsc.VectorSubcoreMesh(core_axis_name="c", subcore_axis_name="s"),
        compiler_params=pltpu.CompilerParams(use_tc_tiling_on_sc=False),
        scratch_types=[
            pltpu.VMEM((EPW,), jnp.int32),
            pltpu.VMEM((EPW,), jnp.int32),
            [pltpu.VMEM((CHUNK, D_HID), jnp.float32)] * NBUF,
            pltpu.VMEM_SHARED((N_NODES, D_HID), jnp.float32),
            [pltpu.SemaphoreType.DMA] * NBUF,
        ],
    )
    partials = sc_segsum(ei, y, zeros)
    p0 = partials[:N_NODES]
    p1 = partials[N_NODES:]

    out = pl.pallas_call(
        _epilogue_body,
        out_shape=jax.ShapeDtypeStruct((N_NODES, D_HID), jnp.float32),
    )(y, p0, p1, b1.reshape(1, D_HID), W2, b2.reshape(1, D_HID))
    return out
